# baseline (speedup 1.0000x reference)
import functools

import jax
import jax.numpy as jnp
from jax import lax
from jax.experimental import pallas as pl
from jax.experimental.pallas import tpu as pltpu
from jax.experimental.pallas import tpu_sc as plsc

VOCAB = 1000
DIM = 64
BATCH = 16384
DIM_PAD = 128

NUM_CORES = 2
NUM_SUBCORES = 16
NW = NUM_CORES * NUM_SUBCORES
NSPLIT = 2
BHALF = BATCH // NSPLIT
B_PER_W = BHALF // NW
IDX_CHUNK = 128
N_CHUNKS = B_PER_W // IDX_CHUNK


def _sc_gather_body(table_hbm, idx_hbm, out_hbm, idx_v, rows_v, sem):
    wid = lax.axis_index("s") * NUM_CORES + lax.axis_index("c")
    base = wid * B_PER_W
    pltpu.sync_copy(idx_hbm.at[pl.ds(wid * N_CHUNKS, N_CHUNKS)], idx_v)
    copies = []
    for j in range(N_CHUNKS):
        copies.append(
            pltpu.async_copy(
                table_hbm.at[idx_v.at[j]],
                rows_v.at[pl.ds(j * IDX_CHUNK, IDX_CHUNK)],
                sem,
            )
        )
    for c in copies:
        c.wait()
    pltpu.sync_copy(rows_v, out_hbm.at[pl.ds(base, B_PER_W)])


def _sc_gather(table, idx2d):
    mesh = plsc.VectorSubcoreMesh(core_axis_name="c", subcore_axis_name="s")
    kern = functools.partial(
        pl.kernel,
        mesh=mesh,
        out_type=jax.ShapeDtypeStruct((BHALF, DIM_PAD), jnp.float32),
        scratch_types=[
            pltpu.VMEM((N_CHUNKS, IDX_CHUNK), jnp.int32),
            pltpu.VMEM((B_PER_W, DIM_PAD), jnp.float32),
            pltpu.SemaphoreType.DMA,
        ],
    )(_sc_gather_body)
    return kern(table, idx2d)


_PROJ_BB = 1024
_BLKS_PER_HALF = BHALF // _PROJ_BB


def _proj_body(w_ref, emb_ref, b_ref, out_ref):
    out_ref[...] = (
        lax.dot_general(
            w_ref[...], emb_ref[...],
            (((1,), (1,)), ((), ())),
            preferred_element_type=jnp.float32,
        )
        + b_ref[...]
    )


def _proj_body_aliased(w_ref, emb_ref, b_ref, prev_ref, out_ref):
    del prev_ref
    _proj_body(w_ref, emb_ref, b_ref, out_ref)


def _tc_project_t_first(w_pad, emb, bcol):
    return pl.pallas_call(
        _proj_body,
        grid=(_BLKS_PER_HALF,),
        in_specs=[
            pl.BlockSpec((VOCAB, DIM_PAD), lambda i: (0, 0)),
            pl.BlockSpec((_PROJ_BB, DIM_PAD), lambda i: (i, 0)),
            pl.BlockSpec((VOCAB, 1), lambda i: (0, 0)),
        ],
        out_specs=pl.BlockSpec((VOCAB, _PROJ_BB), lambda i: (0, i)),
        out_shape=jax.ShapeDtypeStruct((VOCAB, BATCH), jnp.float32),
    )(w_pad, emb, bcol)


def _tc_project_t_second(w_pad, emb, bcol, prev):
    return pl.pallas_call(
        _proj_body_aliased,
        grid=(_BLKS_PER_HALF,),
        in_specs=[
            pl.BlockSpec((VOCAB, DIM_PAD), lambda i: (0, 0)),
            pl.BlockSpec((_PROJ_BB, DIM_PAD), lambda i: (i, 0)),
            pl.BlockSpec((VOCAB, 1), lambda i: (0, 0)),
            pl.BlockSpec(memory_space=pltpu.MemorySpace.HBM),
        ],
        out_specs=pl.BlockSpec((VOCAB, _PROJ_BB), lambda i: (0, i + _BLKS_PER_HALF)),
        out_shape=jax.ShapeDtypeStruct((VOCAB, BATCH), jnp.float32),
        input_output_aliases={3: 0},
    )(w_pad, emb, bcol, prev)


def kernel(inputs, emb_weight, lin_weight, lin_bias):
    idx = inputs.astype(jnp.int32)
    pad = ((0, 0), (0, DIM_PAD - DIM))
    table = jnp.pad(emb_weight, pad)
    w_pad = jnp.pad(lin_weight, pad)
    bcol = lin_bias.reshape(VOCAB, 1)

    idx_a = idx[:BHALF].reshape(BHALF // IDX_CHUNK, IDX_CHUNK)
    idx_b = idx[BHALF:].reshape(BHALF // IDX_CHUNK, IDX_CHUNK)
    emb_a = _sc_gather(table, idx_a)
    emb_b = _sc_gather(table, idx_b)
    out_t = _tc_project_t_first(w_pad, emb_a, bcol)
    out_t = _tc_project_t_second(w_pad, emb_b, bcol, out_t)
    return (out_t.T,)
